# Initial kernel scaffold; baseline (speedup 1.0000x reference)
import jax
import jax.numpy as jnp
from jax.experimental import pallas as pl


def kernel(features, edge_index, gn_gamma, gn_beta, W_fc, attn_h, attn_t, W_out, ffn_gamma, ffn_beta, ff_w1, ff_b1, ff_w2, ff_b2):
    raise NotImplementedError("write your pallas kernel here")


# speedup vs baseline: 1.0121x; 1.0121x over previous
import functools

import jax
import jax.numpy as jnp
from jax.experimental import pallas as pl

N = 10000
E = 160000
D = 256
H = 8
DH = 32
DFF = 1024
ALPHA = 0.15
HOPS = 3
NEG_SLOPE = 0.2

ROWS = 400


def _layer_norm(x, gamma, beta, eps=1e-5):
    mu = jnp.mean(x, axis=-1, keepdims=True)
    var = jnp.mean((x - mu) ** 2, axis=-1, keepdims=True)
    return (x - mu) * jax.lax.rsqrt(var + eps) * gamma + beta


def _pre_body(x_ref, gamma_ref, beta_ref, wfc_t_ref, ah_ref, at_ref,
              feat_ref, eh_ref, et_ref):
    x = x_ref[...]
    h = _layer_norm(x, gamma_ref[...], beta_ref[...])
    feat = jnp.dot(h, wfc_t_ref[...], preferred_element_type=jnp.float32)
    feat_ref[...] = feat
    ft = jnp.tanh(feat)
    eh_ref[...] = jnp.dot(ft, ah_ref[...], preferred_element_type=jnp.float32)
    et_ref[...] = jnp.dot(ft, at_ref[...], preferred_element_type=jnp.float32)


@jax.jit
def _pre(features, gn_gamma, gn_beta, wfc_t, ah, at):
    grid = (N // ROWS,)
    return pl.pallas_call(
        _pre_body,
        grid=grid,
        in_specs=[
            pl.BlockSpec((ROWS, D), lambda i: (i, 0)),
            pl.BlockSpec((D,), lambda i: (0,)),
            pl.BlockSpec((D,), lambda i: (0,)),
            pl.BlockSpec((D, D), lambda i: (0, 0)),
            pl.BlockSpec((D, H), lambda i: (0, 0)),
            pl.BlockSpec((D, H), lambda i: (0, 0)),
        ],
        out_specs=[
            pl.BlockSpec((ROWS, D), lambda i: (i, 0)),
            pl.BlockSpec((ROWS, H), lambda i: (i, 0)),
            pl.BlockSpec((ROWS, H), lambda i: (i, 0)),
        ],
        out_shape=[
            jax.ShapeDtypeStruct((N, D), jnp.float32),
            jax.ShapeDtypeStruct((N, H), jnp.float32),
            jax.ShapeDtypeStruct((N, H), jnp.float32),
        ],
    )(features, gn_gamma, gn_beta, wfc_t, ah, at)


def _post_body(f_ref, x_ref, wout_t_ref, gamma_ref, beta_ref,
               w1t_ref, b1_ref, w2t_ref, b2_ref, out_ref):
    rst = jnp.dot(f_ref[...], wout_t_ref[...],
                  preferred_element_type=jnp.float32) + x_ref[...]
    ff_in = _layer_norm(rst, gamma_ref[...], beta_ref[...])
    hmid = jnp.maximum(
        jnp.dot(ff_in, w1t_ref[...], preferred_element_type=jnp.float32)
        + b1_ref[...], 0.0)
    out_ref[...] = rst + jnp.dot(
        hmid, w2t_ref[...], preferred_element_type=jnp.float32) + b2_ref[...]


@jax.jit
def _post(f, features, wout_t, ffn_gamma, ffn_beta, w1t, b1, w2t, b2):
    grid = (N // ROWS,)
    return pl.pallas_call(
        _post_body,
        grid=grid,
        in_specs=[
            pl.BlockSpec((ROWS, D), lambda i: (i, 0)),
            pl.BlockSpec((ROWS, D), lambda i: (i, 0)),
            pl.BlockSpec((D, D), lambda i: (0, 0)),
            pl.BlockSpec((D,), lambda i: (0,)),
            pl.BlockSpec((D,), lambda i: (0,)),
            pl.BlockSpec((D, DFF), lambda i: (0, 0)),
            pl.BlockSpec((DFF,), lambda i: (0,)),
            pl.BlockSpec((DFF, D), lambda i: (0, 0)),
            pl.BlockSpec((D,), lambda i: (0,)),
        ],
        out_specs=pl.BlockSpec((ROWS, D), lambda i: (i, 0)),
        out_shape=jax.ShapeDtypeStruct((N, D), jnp.float32),
    )(f, features, wout_t, ffn_gamma, ffn_beta, w1t, b1, w2t, b2)


def kernel(features, edge_index, gn_gamma, gn_beta, W_fc, attn_h, attn_t,
           W_out, ffn_gamma, ffn_beta, ff_w1, ff_b1, ff_w2, ff_b2):
    ah = jnp.zeros((D, H), jnp.float32)
    at = jnp.zeros((D, H), jnp.float32)
    cols = jnp.arange(D, dtype=jnp.int32)
    head_of_col = cols // DH
    ah = ah.at[cols, head_of_col].set(attn_h.reshape(D))
    at = at.at[cols, head_of_col].set(attn_t.reshape(D))

    feat, eh, et = _pre(features, gn_gamma, gn_beta, W_fc.T, ah, at)

    src = edge_index[0]
    dst = edge_index[1]
    e = eh[src] + et[dst]
    e = jnp.where(e > 0, e, NEG_SLOPE * e)
    p = jnp.exp(e)
    s = jax.ops.segment_sum(p, dst, num_segments=N)
    a = (p / (s[dst] + 1e-16))[..., None]

    feat0 = feat.reshape(N, H, DH)
    f = feat0
    for _ in range(HOPS):
        m = f[src] * a
        agg = jax.ops.segment_sum(m, dst, num_segments=N)
        f = (1.0 - ALPHA) * agg + ALPHA * feat0

    return _post(f.reshape(N, D), features, W_out.T, ffn_gamma, ffn_beta,
                 ff_w1.T, ff_b1, ff_w2.T, ff_b2)
